# Initial kernel scaffold; baseline (speedup 1.0000x reference)
import jax
import jax.numpy as jnp
from jax.experimental import pallas as pl


def kernel(token, week, hour, duration, token_w, week_w, hour_w, dur_w):
    raise NotImplementedError("write your pallas kernel here")


# speedup vs baseline: 2.1677x; 2.1677x over previous
import functools

import jax
import jax.numpy as jnp
from jax import lax
from jax.experimental import pallas as pl
from jax.experimental.pallas import tpu as pltpu
from jax.experimental.pallas import tpu_sc as plsc

B, L = 4096, 200
N = B * L
TOKEN_D = 64
OUT_D = 112
NC, NS = 2, 16
NW = NC * NS
ROWS_PER_W = N // NW
C = 256
N_CHUNKS = ROWS_PER_W // C
G = 128


def _body(tok_hbm, wk_hbm, hr_hbm, du_hbm,
          tokw_hbm, wkw_hbm, hrw_hbm, duw_hbm,
          out_hbm,
          tok_idx_v, wk_idx_v, hr_idx_v, du_idx_v,
          wtab_v, htab_v, dtab_v,
          tok_rows_v, out_v, sem, gsem):
    wid = lax.axis_index("s") * NC + lax.axis_index("c")
    iota = jax.lax.iota(jnp.int32, 16)

    pltpu.sync_copy(wkw_hbm, wtab_v)
    pltpu.sync_copy(hrw_hbm, htab_v)
    pltpu.sync_copy(duw_hbm, dtab_v)

    def chunk_body(c, carry):
        base = wid * ROWS_PER_W + c * C
        pltpu.sync_copy(tok_hbm.at[pl.ds(base, C)], tok_idx_v)
        pltpu.sync_copy(wk_hbm.at[pl.ds(base, C)], wk_idx_v)
        pltpu.sync_copy(hr_hbm.at[pl.ds(base, C)], hr_idx_v)
        pltpu.sync_copy(du_hbm.at[pl.ds(base, C)], du_idx_v)

        copies = []
        for s in range(C // G):
            copies.append(pltpu.async_copy(
                tokw_hbm.at[tok_idx_v.at[pl.ds(s * G, G)]],
                tok_rows_v.at[pl.ds(s * G, G)],
                gsem))
        def group_body(g, carry2):
            rowv = g * 16 + iota
            widx = wk_idx_v[pl.ds(g * 16, 16)] * 16
            hidx = hr_idx_v[pl.ds(g * 16, 16)] * 16
            didx = du_idx_v[pl.ds(g * 16, 16)] * 16
            for k in range(16):
                vals = plsc.load_gather(wtab_v, [widx + k])
                plsc.store_scatter(out_v, [rowv, jnp.full((16,), 64 + k, jnp.int32)], vals)
                vals = plsc.load_gather(htab_v, [hidx + k])
                plsc.store_scatter(out_v, [rowv, jnp.full((16,), 80 + k, jnp.int32)], vals)
                vals = plsc.load_gather(dtab_v, [didx + k])
                plsc.store_scatter(out_v, [rowv, jnp.full((16,), 96 + k, jnp.int32)], vals)
            return carry2
        lax.fori_loop(0, C // 16, group_body, 0, unroll=False)
        for cp in copies:
            cp.wait()

        def tok_body(g, carry2):
            rowv = g * 16 + iota
            for k in range(64):
                kv = jnp.full((16,), k, jnp.int32)
                vals = plsc.load_gather(tok_rows_v, [rowv, kv])
                plsc.store_scatter(out_v, [rowv, kv], vals)
            return carry2
        lax.fori_loop(0, C // 16, tok_body, 0, unroll=False)

        pltpu.sync_copy(out_v, out_hbm.at[pl.ds(base, C)])
        return carry

    lax.fori_loop(0, N_CHUNKS, chunk_body, 0, unroll=False)


@jax.jit
def _launch(tok, wk, hr, du, tokw, wkw, hrw, duw):
    mesh = plsc.VectorSubcoreMesh(core_axis_name="c", subcore_axis_name="s")
    kfn = functools.partial(
        pl.kernel,
        mesh=mesh,
        compiler_params=pltpu.CompilerParams(needs_layout_passes=False),
        out_type=jax.ShapeDtypeStruct((N, OUT_D), jnp.float32),
        scratch_types=[
            pltpu.VMEM((C,), jnp.int32),
            pltpu.VMEM((C,), jnp.int32),
            pltpu.VMEM((C,), jnp.int32),
            pltpu.VMEM((C,), jnp.int32),
            pltpu.VMEM((7 * 16,), jnp.float32),
            pltpu.VMEM((24 * 16,), jnp.float32),
            pltpu.VMEM((24 * 16,), jnp.float32),
            pltpu.VMEM((C, 128), jnp.float32),
            pltpu.VMEM((C, OUT_D), jnp.float32),
            pltpu.SemaphoreType.DMA,
            pltpu.SemaphoreType.DMA,
        ],
    )(_body)
    return kfn(tok, wk, hr, du, tokw, wkw, hrw, duw)


def kernel(token, week, hour, duration, token_w, week_w, hour_w, dur_w):
    token_w = jnp.pad(token_w, ((0, 0), (0, 128 - TOKEN_D)))
    tok = token.reshape(-1).astype(jnp.int32)
    wk = week.reshape(-1).astype(jnp.int32)
    hr = hour.reshape(-1).astype(jnp.int32)
    du = duration.reshape(-1).astype(jnp.int32)
    out = _launch(tok, wk, hr, du, token_w,
                  week_w.reshape(-1), hour_w.reshape(-1), dur_w.reshape(-1))
    return out.reshape(B, L, OUT_D)


# speedup vs baseline: 4.4396x; 2.0480x over previous
import functools

import jax
import jax.numpy as jnp
from jax import lax
from jax.experimental import pallas as pl
from jax.experimental.pallas import tpu as pltpu
from jax.experimental.pallas import tpu_sc as plsc

B, L = 4096, 200
N = B * L
TOKEN_D = 64
OUT_D = 112
PAD_D = 128
NC, NS = 2, 16
NW = NC * NS
ROWS_PER_W = N // NW
C = 256
N_CHUNKS = ROWS_PER_W // C
G = 128


def _body(tok_hbm, wk_hbm, hr_hbm, du_hbm,
          tokw_hbm, wkw_hbm, hrw_hbm, duw_hbm,
          out_hbm,
          tok_idx_v, wk_idx_v, hr_idx_v, du_idx_v,
          wtab_v, htab_v, dtab_v,
          tok_rows_v, out_v, sem, gsem):
    wid = lax.axis_index("s") * NC + lax.axis_index("c")
    iota = jax.lax.iota(jnp.int32, 16)

    pltpu.sync_copy(wkw_hbm, wtab_v)
    pltpu.sync_copy(hrw_hbm, htab_v)
    pltpu.sync_copy(duw_hbm, dtab_v)

    def chunk_body(c, carry):
        base = wid * ROWS_PER_W + c * C
        pltpu.sync_copy(tok_hbm.at[pl.ds(base, C)], tok_idx_v)
        pltpu.sync_copy(wk_hbm.at[pl.ds(base, C)], wk_idx_v)
        pltpu.sync_copy(hr_hbm.at[pl.ds(base, C)], hr_idx_v)
        pltpu.sync_copy(du_hbm.at[pl.ds(base, C)], du_idx_v)

        copies = []
        for s in range(C // G):
            copies.append(pltpu.async_copy(
                tokw_hbm.at[tok_idx_v.at[pl.ds(s * G, G)]],
                tok_rows_v.at[pl.ds(s * G, G)],
                gsem))
        for cp in copies:
            cp.wait()

        def row_body(i, carry2):
            bidx = jnp.full((16,), i, jnp.int32)
            widx = plsc.load_gather(wk_idx_v, [bidx]) * 16
            hidx = plsc.load_gather(hr_idx_v, [bidx]) * 16
            didx = plsc.load_gather(du_idx_v, [bidx]) * 16
            for k in range(4):
                out_v[i, pl.ds(k * 16, 16)] = tok_rows_v[i, pl.ds(k * 16, 16)]
            out_v[i, pl.ds(64, 16)] = plsc.load_gather(wtab_v, [widx + iota])
            out_v[i, pl.ds(80, 16)] = plsc.load_gather(htab_v, [hidx + iota])
            out_v[i, pl.ds(96, 16)] = plsc.load_gather(dtab_v, [didx + iota])
            return carry2
        lax.fori_loop(0, C, row_body, 0, unroll=False)

        pltpu.sync_copy(out_v, out_hbm.at[pl.ds(base, C)])
        return carry

    lax.fori_loop(0, N_CHUNKS, chunk_body, 0, unroll=False)


@jax.jit
def _launch(tok, wk, hr, du, tokw, wkw, hrw, duw):
    mesh = plsc.VectorSubcoreMesh(core_axis_name="c", subcore_axis_name="s")
    kfn = functools.partial(
        pl.kernel,
        mesh=mesh,
        compiler_params=pltpu.CompilerParams(needs_layout_passes=False),
        out_type=jax.ShapeDtypeStruct((N, OUT_D), jnp.float32),
        scratch_types=[
            pltpu.VMEM((C,), jnp.int32),
            pltpu.VMEM((C,), jnp.int32),
            pltpu.VMEM((C,), jnp.int32),
            pltpu.VMEM((C,), jnp.int32),
            pltpu.VMEM((7 * 16,), jnp.float32),
            pltpu.VMEM((24 * 16,), jnp.float32),
            pltpu.VMEM((24 * 16,), jnp.float32),
            pltpu.VMEM((C, PAD_D), jnp.float32),
            pltpu.VMEM((C, OUT_D), jnp.float32),
            pltpu.SemaphoreType.DMA,
            pltpu.SemaphoreType.DMA,
        ],
    )(_body)
    return kfn(tok, wk, hr, du, tokw, wkw, hrw, duw)


def kernel(token, week, hour, duration, token_w, week_w, hour_w, dur_w):
    token_w = jnp.pad(token_w, ((0, 0), (0, PAD_D - TOKEN_D)))
    tok = token.reshape(-1).astype(jnp.int32)
    wk = week.reshape(-1).astype(jnp.int32)
    hr = hour.reshape(-1).astype(jnp.int32)
    du = duration.reshape(-1).astype(jnp.int32)
    out = _launch(tok, wk, hr, du, token_w,
                  week_w.reshape(-1), hour_w.reshape(-1), dur_w.reshape(-1))
    return out.reshape(B, L, OUT_D)


# speedup vs baseline: 4.9475x; 1.1144x over previous
import functools

import jax
import jax.numpy as jnp
from jax import lax
from jax.experimental import pallas as pl
from jax.experimental.pallas import tpu as pltpu
from jax.experimental.pallas import tpu_sc as plsc

B, L = 4096, 200
N = B * L
NUM_V = 1000000
TOKEN_D = 64
OUT_D = 112
PAD_D = 128
NC, NS = 2, 16
NW = NC * NS
ROWS_PER_W = N // NW
C = 200
N_CHUNKS = ROWS_PER_W // C
G_SPLITS = ((0, 128), (128, 72))


def _body(tok_hbm, wk_hbm, hr_hbm, du_hbm,
          tokw_hbm, wkw_hbm, hrw_hbm, duw_hbm,
          out_hbm,
          ti0, ti1, wk_idx_v, hr_idx_v, du_idx_v,
          wtab_v, htab_v, dtab_v,
          tr0, tr1, o0, o1, gsem, osem):
    wid = lax.axis_index("s") * NC + lax.axis_index("c")
    iota = jax.lax.iota(jnp.int32, 16)
    w0 = wid * ROWS_PER_W

    pltpu.sync_copy(wkw_hbm, wtab_v)
    pltpu.sync_copy(hrw_hbm, htab_v)
    pltpu.sync_copy(duw_hbm, dtab_v)

    def fire_gathers(c, ti, tr):
        pltpu.sync_copy(tok_hbm.at[pl.ds(w0 + c * C, C)], ti.at[pl.ds(0, C)])
        for off, g in G_SPLITS:
            pltpu.async_copy(
                tokw_hbm.at[ti.at[pl.ds(off, g)]],
                tr.at[pl.ds(off, g)],
                gsem)

    def wait_gathers(tr):
        for off, g in G_SPLITS:
            pltpu.make_async_copy(
                tokw_hbm.at[pl.ds(0, g)],
                tr.at[pl.ds(off, g)],
                gsem).wait()

    def wait_out(o):
        pltpu.make_async_copy(o, out_hbm.at[pl.ds(0, C)], osem).wait()

    fire_gathers(0, ti0, tr0)

    def step(c, ti_n, tr_n, tr, o):
        base = w0 + c * C
        pltpu.sync_copy(wk_hbm.at[pl.ds(base, C)], wk_idx_v)
        pltpu.sync_copy(hr_hbm.at[pl.ds(base, C)], hr_idx_v)
        pltpu.sync_copy(du_hbm.at[pl.ds(base, C)], du_idx_v)

        @pl.when(c >= 2)
        def _():
            wait_out(o)

        wait_gathers(tr)

        def row_body(i, carry2):
            bidx = jnp.full((16,), i, jnp.int32)
            widx = plsc.load_gather(wk_idx_v, [bidx]) * 16
            hidx = plsc.load_gather(hr_idx_v, [bidx]) * 16
            didx = plsc.load_gather(du_idx_v, [bidx]) * 16
            for k in range(4):
                o[i, pl.ds(k * 16, 16)] = tr[i, pl.ds(k * 16, 16)]
            o[i, pl.ds(64, 16)] = plsc.load_gather(wtab_v, [widx + iota])
            o[i, pl.ds(80, 16)] = plsc.load_gather(htab_v, [hidx + iota])
            o[i, pl.ds(96, 16)] = plsc.load_gather(dtab_v, [didx + iota])
            return carry2
        lax.fori_loop(0, C, row_body, 0, unroll=False)

        pltpu.async_copy(o, out_hbm.at[pl.ds(base, C)], osem)

        @pl.when(c + 1 < N_CHUNKS)
        def _():
            fire_gathers(c + 1, ti_n, tr_n)

    def pair_body(h, carry):
        step(2 * h, ti1, tr1, tr0, o0)
        step(2 * h + 1, ti0, tr0, tr1, o1)
        return carry

    lax.fori_loop(0, N_CHUNKS // 2, pair_body, 0, unroll=False)

    wait_out(o0)
    wait_out(o1)


@jax.jit
def _launch(tok, wk, hr, du, tokw, wkw, hrw, duw):
    mesh = plsc.VectorSubcoreMesh(core_axis_name="c", subcore_axis_name="s")
    kfn = functools.partial(
        pl.kernel,
        mesh=mesh,
        compiler_params=pltpu.CompilerParams(needs_layout_passes=False),
        out_type=jax.ShapeDtypeStruct((N, OUT_D), jnp.float32),
        scratch_types=[
            pltpu.VMEM((C,), jnp.int32),
            pltpu.VMEM((C,), jnp.int32),
            pltpu.VMEM((C,), jnp.int32),
            pltpu.VMEM((C,), jnp.int32),
            pltpu.VMEM((C,), jnp.int32),
            pltpu.VMEM((7 * 16,), jnp.float32),
            pltpu.VMEM((24 * 16,), jnp.float32),
            pltpu.VMEM((24 * 16,), jnp.float32),
            pltpu.VMEM((C, PAD_D), jnp.float32),
            pltpu.VMEM((C, PAD_D), jnp.float32),
            pltpu.VMEM((C, OUT_D), jnp.float32),
            pltpu.VMEM((C, OUT_D), jnp.float32),
            pltpu.SemaphoreType.DMA,
            pltpu.SemaphoreType.DMA,
        ],
    )(_body)
    return kfn(tok, wk, hr, du, tokw, wkw, hrw, duw)


def kernel(token, week, hour, duration, token_w, week_w, hour_w, dur_w):
    token_w = jnp.pad(token_w, ((0, 0), (0, PAD_D - TOKEN_D)))
    tok = token.reshape(-1).astype(jnp.int32)
    wk = week.reshape(-1).astype(jnp.int32)
    hr = hour.reshape(-1).astype(jnp.int32)
    du = duration.reshape(-1).astype(jnp.int32)
    out = _launch(tok, wk, hr, du, token_w,
                  week_w.reshape(-1), hour_w.reshape(-1), dur_w.reshape(-1))
    return out.reshape(B, L, OUT_D)
